# Initial kernel scaffold; baseline (speedup 1.0000x reference)
import jax
import jax.numpy as jnp
from jax.experimental import pallas as pl


def kernel(x, batch):
    raise NotImplementedError("write your pallas kernel here")


# speedup vs baseline: 4.6126x; 4.6126x over previous
import functools

import jax
import jax.numpy as jnp
from jax import lax
from jax.experimental import pallas as pl
from jax.experimental.pallas import tpu as pltpu
from jax.experimental.pallas import tpu_sc as plsc

N = 100000
D = 128
S = 512
C = 128
NFULL = N // C
REM = N - NFULL * C
NW = 32
CHUNKS_PER_W = (NFULL + 1 + NW - 1) // NW
RPT = S // 16


def _sc_pool(x, batch, ones_hbm, zeros_hbm):
    mesh = plsc.VectorSubcoreMesh(core_axis_name="c", subcore_axis_name="s")

    @functools.partial(
        pl.kernel,
        mesh=mesh,
        out_type=[
            jax.ShapeDtypeStruct((2 * S, D), jnp.float32),
            jax.ShapeDtypeStruct((2 * S, D), jnp.float32),
        ],
        scratch_types=[
            pltpu.VMEM((C,), jnp.int32),
            pltpu.VMEM((C, D), jnp.float32),
            pltpu.VMEM((C, D), jnp.float32),
            pltpu.VMEM((REM,), jnp.int32),
            pltpu.VMEM((RPT, D), jnp.float32),
            pltpu.VMEM_SHARED((S, D), jnp.float32),
            pltpu.VMEM_SHARED((S, D), jnp.float32),
        ],
    )
    def pool(x_hbm, b_hbm, ones_h, zeros_h, out_hbm, cnt_hbm,
             idx_v, xbuf, ones_v, idx_r, stage, acc_sh, cnt_sh):
        cid = lax.axis_index("c")
        sid = lax.axis_index("s")
        wid = sid * 2 + cid

        pltpu.sync_copy(ones_h, ones_v)
        pltpu.sync_copy(zeros_h, stage)
        row0 = sid * RPT
        pltpu.sync_copy(stage, acc_sh.at[pl.ds(row0, RPT)])
        pltpu.sync_copy(stage, cnt_sh.at[pl.ds(row0, RPT)])
        plsc.subcore_barrier()

        def body(j, carry):
            k = wid + NW * j

            @pl.when(k < NFULL)
            def _full():
                base = k * C
                pltpu.sync_copy(b_hbm.at[pl.ds(base, C)], idx_v)
                pltpu.sync_copy(x_hbm.at[pl.ds(base, C)], xbuf)
                pltpu.sync_copy(xbuf, acc_sh.at[idx_v], add=True)
                pltpu.sync_copy(ones_v, cnt_sh.at[idx_v], add=True)

            @pl.when(k == NFULL)
            def _partial():
                base = NFULL * C
                pltpu.sync_copy(b_hbm.at[pl.ds(base, REM)], idx_r)
                pltpu.sync_copy(x_hbm.at[pl.ds(base, REM)], stage)
                pltpu.sync_copy(stage, acc_sh.at[idx_r], add=True)
                pltpu.sync_copy(ones_v.at[pl.ds(0, REM)], cnt_sh.at[idx_r],
                                add=True)

            return carry

        lax.fori_loop(0, CHUNKS_PER_W, body, 0)
        plsc.subcore_barrier()

        out_row = cid * S + row0
        pltpu.sync_copy(acc_sh.at[pl.ds(row0, RPT)], stage)
        pltpu.sync_copy(stage, out_hbm.at[pl.ds(out_row, RPT)])
        pltpu.sync_copy(cnt_sh.at[pl.ds(row0, RPT)], stage)
        pltpu.sync_copy(stage, cnt_hbm.at[pl.ds(out_row, RPT)])

    return pool(x, batch, ones_hbm, zeros_hbm)


def _merge_body(p_ref, c_ref, o_ref):
    p = p_ref[0:S, :] + p_ref[S:2 * S, :]
    c = c_ref[0:S, 0:1] + c_ref[S:2 * S, 0:1]
    o_ref[...] = p / jnp.maximum(c, 1.0)


def kernel(x, batch):
    batch = batch.astype(jnp.int32)
    ones_hbm = jnp.ones((C, D), jnp.float32)
    zeros_hbm = jnp.zeros((RPT, D), jnp.float32)
    partial, cnt = _sc_pool(x, batch, ones_hbm, zeros_hbm)
    out = pl.pallas_call(
        _merge_body,
        out_shape=jax.ShapeDtypeStruct((S, D), jnp.float32),
    )(partial, cnt)
    return out


# speedup vs baseline: 6.5012x; 1.4095x over previous
import functools

import jax
import jax.numpy as jnp
from jax import lax
from jax.experimental import pallas as pl
from jax.experimental.pallas import tpu as pltpu
from jax.experimental.pallas import tpu_sc as plsc

N = 100000
D = 128
S = 512
C = 128
NFULL = N // C
REM = N - NFULL * C
NW = 32
NPAIR = 12
RPT = S // 16


def _sc_pool(x, batch, ones_hbm, zeros_hbm):
    mesh = plsc.VectorSubcoreMesh(core_axis_name="c", subcore_axis_name="s")

    @functools.partial(
        pl.kernel,
        mesh=mesh,
        out_type=[
            jax.ShapeDtypeStruct((2 * S, D), jnp.float32),
            jax.ShapeDtypeStruct((2 * S, D), jnp.float32),
        ],
        scratch_types=[
            pltpu.VMEM((C,), jnp.int32),
            pltpu.VMEM((C, D), jnp.float32),
            pltpu.VMEM((C,), jnp.int32),
            pltpu.VMEM((C, D), jnp.float32),
            pltpu.VMEM((C, D), jnp.float32),
            pltpu.VMEM((REM,), jnp.int32),
            pltpu.VMEM((RPT, D), jnp.float32),
            pltpu.VMEM_SHARED((S, D), jnp.float32),
            pltpu.VMEM_SHARED((S, D), jnp.float32),
            pltpu.SemaphoreType.DMA,
            pltpu.SemaphoreType.DMA,
            pltpu.SemaphoreType.DMA,
            pltpu.SemaphoreType.DMA,
        ],
    )
    def pool(x_hbm, b_hbm, ones_h, zeros_h, out_hbm, cnt_hbm,
             idx0, xb0, idx1, xb1, ones_v, idx_r, stage, acc_sh, cnt_sh,
             sg0, sg1, ss0, ss1):
        cid = lax.axis_index("c")
        sid = lax.axis_index("s")
        wid = sid * 2 + cid

        pltpu.sync_copy(ones_h, ones_v)
        pltpu.sync_copy(zeros_h, stage)
        row0 = sid * RPT
        pltpu.sync_copy(stage, acc_sh.at[pl.ds(row0, RPT)])
        pltpu.sync_copy(stage, cnt_sh.at[pl.ds(row0, RPT)])
        plsc.subcore_barrier()

        def gather(k, idx_v, xbuf, sem):
            base = k * C
            pltpu.async_copy(b_hbm.at[pl.ds(base, C)], idx_v, sem)
            pltpu.async_copy(x_hbm.at[pl.ds(base, C)], xbuf, sem)

        def wait_gather(idx_v, xbuf, sem):
            pltpu.make_async_copy(b_hbm.at[pl.ds(0, C)], idx_v, sem).wait()
            pltpu.make_async_copy(x_hbm.at[pl.ds(0, C)], xbuf, sem).wait()

        def scatter(idx_v, xbuf, sem):
            pltpu.async_copy(xbuf, acc_sh.at[idx_v], sem, add=True)
            pltpu.async_copy(ones_v, cnt_sh.at[idx_v], sem, add=True)

        def wait_scatter(idx_v, xbuf, sem):
            pltpu.make_async_copy(xbuf, acc_sh.at[idx_v], sem).wait()
            pltpu.make_async_copy(ones_v, cnt_sh.at[idx_v], sem).wait()

        gather(wid, idx0, xb0, sg0)

        def body(jj, carry):
            k0 = wid + NW * (2 * jj)
            k1 = k0 + NW
            wait_gather(idx0, xb0, sg0)
            scatter(idx0, xb0, ss0)
            gather(k1, idx1, xb1, sg1)
            wait_scatter(idx0, xb0, ss0)
            wait_gather(idx1, xb1, sg1)
            scatter(idx1, xb1, ss1)

            @pl.when(jj < NPAIR - 1)
            def _():
                gather(k1 + NW, idx0, xb0, sg0)

            wait_scatter(idx1, xb1, ss1)
            return carry

        lax.fori_loop(0, NPAIR, body, 0)

        k = wid + NW * (2 * NPAIR)

        @pl.when(k < NFULL)
        def _full():
            base = k * C
            pltpu.sync_copy(b_hbm.at[pl.ds(base, C)], idx0)
            pltpu.sync_copy(x_hbm.at[pl.ds(base, C)], xb0)
            pltpu.sync_copy(xb0, acc_sh.at[idx0], add=True)
            pltpu.sync_copy(ones_v, cnt_sh.at[idx0], add=True)

        @pl.when(k == NFULL)
        def _partial():
            base = NFULL * C
            pltpu.sync_copy(b_hbm.at[pl.ds(base, REM)], idx_r)
            pltpu.sync_copy(x_hbm.at[pl.ds(base, REM)], stage)
            pltpu.sync_copy(stage, acc_sh.at[idx_r], add=True)
            pltpu.sync_copy(ones_v.at[pl.ds(0, REM)], cnt_sh.at[idx_r],
                            add=True)

        plsc.subcore_barrier()

        out_row = cid * S + row0
        pltpu.sync_copy(acc_sh.at[pl.ds(row0, RPT)], stage)
        pltpu.sync_copy(stage, out_hbm.at[pl.ds(out_row, RPT)])
        pltpu.sync_copy(cnt_sh.at[pl.ds(row0, RPT)], stage)
        pltpu.sync_copy(stage, cnt_hbm.at[pl.ds(out_row, RPT)])

    return pool(x, batch, ones_hbm, zeros_hbm)


def _merge_body(p_ref, c_ref, o_ref):
    p = p_ref[0:S, :] + p_ref[S:2 * S, :]
    c = c_ref[0:S, 0:1] + c_ref[S:2 * S, 0:1]
    o_ref[...] = p / jnp.maximum(c, 1.0)


def kernel(x, batch):
    batch = batch.astype(jnp.int32)
    ones_hbm = jnp.ones((C, D), jnp.float32)
    zeros_hbm = jnp.zeros((RPT, D), jnp.float32)
    partial, cnt = _sc_pool(x, batch, ones_hbm, zeros_hbm)
    out = pl.pallas_call(
        _merge_body,
        out_shape=jax.ShapeDtypeStruct((S, D), jnp.float32),
    )(partial, cnt)
    return out
